# Initial kernel scaffold; baseline (speedup 1.0000x reference)
import jax
import jax.numpy as jnp
from jax.experimental import pallas as pl


def kernel(indices, adjustment):
    raise NotImplementedError("write your pallas kernel here")


# speedup vs baseline: 1.0986x; 1.0986x over previous
import functools

import jax
import jax.numpy as jnp
from jax import lax
from jax.experimental import pallas as pl
from jax.experimental.pallas import tpu as pltpu
from jax.experimental.pallas import tpu_sc as plsc

_NUM_CAMERAS = 100000
_BATCH = 16384

_NC = 2
_NS = 16
_NW = _NC * _NS
_B_PER_W = _BATCH // _NW
_CHUNK = 128
_NCHUNK = _B_PER_W // _CHUNK


@functools.partial(
    pl.kernel,
    mesh=plsc.VectorSubcoreMesh(core_axis_name="c", subcore_axis_name="s"),
    out_type=jax.ShapeDtypeStruct((_NW, _NCHUNK, _CHUNK), jnp.float32),
    scratch_types=[
        pltpu.VMEM((_NCHUNK, _CHUNK), jnp.int32),
        pltpu.VMEM((_NCHUNK, _CHUNK), jnp.float32),
        pltpu.SemaphoreType.DMA,
    ],
)
def _sc_gather(idx_hbm, table_hbm, out_hbm, idx_v, vals_v, sem):
    wid = lax.axis_index("s") * _NC + lax.axis_index("c")
    pltpu.sync_copy(idx_hbm.at[wid], idx_v)
    copies = [
        pltpu.async_copy(table_hbm.at[idx_v.at[j]], vals_v.at[j], sem)
        for j in range(_NCHUNK)
    ]
    for c in copies:
        c.wait()
    pltpu.sync_copy(vals_v, out_hbm.at[wid])


def kernel(indices, adjustment):
    idx = indices.astype(jnp.int32).reshape(_NW, _NCHUNK, _CHUNK)
    out = _sc_gather(idx, adjustment)
    return out.reshape(_BATCH)


# speedup vs baseline: 1.1042x; 1.0051x over previous
import functools

import jax
import jax.numpy as jnp
from jax import lax
from jax.experimental import pallas as pl
from jax.experimental.pallas import tpu as pltpu
from jax.experimental.pallas import tpu_sc as plsc

_NUM_CAMERAS = 100000
_BATCH = 16384

_NC = 2
_NS = 16
_NW = _NC * _NS
_B_PER_W = _BATCH // _NW
_CHUNK = 128
_NCHUNK = _B_PER_W // _CHUNK


@functools.partial(
    pl.kernel,
    mesh=plsc.VectorSubcoreMesh(core_axis_name="c", subcore_axis_name="s"),
    out_type=jax.ShapeDtypeStruct((_NW, _NCHUNK, _CHUNK), jnp.float32),
    scratch_types=[
        pltpu.VMEM((_NCHUNK, _CHUNK), jnp.int32),
        pltpu.VMEM((_NCHUNK, _CHUNK), jnp.float32),
        pltpu.SemaphoreType.DMA,
        pltpu.SemaphoreType.DMA,
        pltpu.SemaphoreType.DMA,
        pltpu.SemaphoreType.DMA,
        pltpu.SemaphoreType.DMA,
        pltpu.SemaphoreType.DMA,
        pltpu.SemaphoreType.DMA,
        pltpu.SemaphoreType.DMA,
        pltpu.SemaphoreType.DMA,
    ],
)
def _sc_gather(idx_hbm, table_hbm, out_hbm, idx_v, vals_v,
               si0, si1, si2, si3, sg0, sg1, sg2, sg3, so):
    si = (si0, si1, si2, si3)
    sg = (sg0, sg1, sg2, sg3)
    wid = lax.axis_index("s") * _NC + lax.axis_index("c")
    blk = idx_hbm.at[wid]
    hi = [pltpu.async_copy(blk.at[j], idx_v.at[j], si[j]) for j in range(_NCHUNK)]
    hg = []
    for j in range(_NCHUNK):
        hi[j].wait()
        hg.append(pltpu.async_copy(table_hbm.at[idx_v.at[j]], vals_v.at[j], sg[j]))
    oblk = out_hbm.at[wid]
    ho = []
    for j in range(_NCHUNK):
        hg[j].wait()
        ho.append(pltpu.async_copy(vals_v.at[j], oblk.at[j], so))
    for c in ho:
        c.wait()


def kernel(indices, adjustment):
    idx = indices.astype(jnp.int32).reshape(_NW, _NCHUNK, _CHUNK)
    out = _sc_gather(idx, adjustment)
    return out.reshape(_BATCH)
